# baseline (speedup 1.0000x reference)
import jax
import jax.numpy as jnp
from jax import lax
from jax.experimental import pallas as pl
from jax.experimental.pallas import tpu as pltpu
from jax.experimental.pallas import tpu_sc as plsc

NUM_USERS = 100000
NUM_ITEMS = 1000
BATCH = 16384
HIST = 200

NC, NS, L = 2, 16, 16
NW = NC * NS
BPW = BATCH // NW
C = 32
NCHUNK = BPW // C
VPC = C * HIST // L


def _body(user_hbm, items_hbm, rowid_hbm, colid_hbm, score_hbm, out_hbm,
          uid_v, items_v, rowid_v, colid_v, rows_v, out_v, sem):
    wid = lax.axis_index("s") * NC + lax.axis_index("c")
    pltpu.sync_copy(rowid_hbm, rowid_v)
    pltpu.sync_copy(colid_hbm, colid_v)

    def chunk(g, carry):
        base = wid * BPW + g * C
        pltpu.sync_copy(user_hbm.at[pl.ds(base, C)], uid_v)
        uvecs = [uid_v[pl.ds(k * L, L)] for k in range(C // L)]
        copies = [
            pltpu.async_copy(score_hbm.at[uvecs[i // L][i % L], :],
                             rows_v.at[i, :], sem)
            for i in range(C)
        ]
        pltpu.sync_copy(items_hbm.at[pl.ds(base, C), :], items_v)
        for cp in copies:
            cp.wait()

        def step(v, c2):
            sl = pl.ds(v * L, L)
            row = rowid_v[sl]
            col = colid_v[sl]
            item = plsc.load_gather(items_v, [row, col])
            vals = plsc.load_gather(rows_v, [row, item])
            plsc.store_scatter(out_v, [row, col], vals)
            return c2

        lax.fori_loop(0, VPC, step, 0)
        pltpu.sync_copy(out_v, out_hbm.at[pl.ds(base, C), :])
        return carry

    lax.fori_loop(0, NCHUNK, chunk, 0)


@jax.jit
def _run(batch_user, batch_items, rowid, colid, score_mat):
    mesh = plsc.VectorSubcoreMesh(core_axis_name="c", subcore_axis_name="s")
    f = pl.kernel(
        _body,
        out_type=jax.ShapeDtypeStruct((BATCH, HIST), jnp.float32),
        mesh=mesh,
        compiler_params=pltpu.CompilerParams(use_tc_tiling_on_sc=True,
                                             needs_layout_passes=False),
        scratch_types=[
            pltpu.VMEM((C,), jnp.int32),
            pltpu.VMEM((C, HIST), jnp.int32),
            pltpu.VMEM((C * HIST,), jnp.int32),
            pltpu.VMEM((C * HIST,), jnp.int32),
            pltpu.VMEM((C, NUM_ITEMS), jnp.float32),
            pltpu.VMEM((C, HIST), jnp.float32),
            pltpu.SemaphoreType.DMA,
        ],
    )
    return f(batch_user, batch_items, rowid, colid, score_mat)


def kernel(batch_user, batch_items, score_mat):
    p = jnp.arange(C * HIST, dtype=jnp.int32)
    return _run(batch_user, batch_items, p // HIST, p % HIST, score_mat)


# speedup vs baseline: 1.3778x; 1.3778x over previous
import jax
import jax.numpy as jnp
from jax import lax
from jax.experimental import pallas as pl
from jax.experimental.pallas import tpu as pltpu
from jax.experimental.pallas import tpu_sc as plsc

NUM_USERS = 100000
NUM_ITEMS = 1000
BATCH = 16384
HIST = 200

NC, NS, L = 2, 16, 16
NW = NC * NS
CCOLS = 128
NCHUNK = BATCH // (NW * CCOLS)
SUB = 32
NSUB = CCOLS // SUB
VPS = SUB * HIST // L


def _issue(user_hbm, score_hbm, uid_ref, rows_ref, sem, base):
    pltpu.sync_copy(user_hbm.at[pl.ds(base, SUB)], uid_ref)
    uvecs = [uid_ref[pl.ds(k * L, L)] for k in range(SUB // L)]
    for i in range(SUB):
        pltpu.async_copy(score_hbm.at[uvecs[i // L][i % L], :],
                         rows_ref.at[i, :], sem)


def _drain(score_hbm, rows_ref, sem):
    pltpu.make_async_copy(score_hbm.at[pl.ds(0, SUB), :], rows_ref,
                          sem).wait()


def _body(user_hbm, items_hbm, score_hbm, out_hbm,
          uid_a, uid_b, items_v, out_v, rows_a, rows_b, sem_a, sem_b):
    wid = lax.axis_index("s") * NC + lax.axis_index("c")
    col00 = wid * (NCHUNK * CCOLS)
    bufs = [(uid_a, rows_a, sem_a), (uid_b, rows_b, sem_b)]
    _issue(user_hbm, score_hbm, uid_a, rows_a, sem_a, col00)

    def chunk(g, carry):
        col0 = col00 + g * CCOLS
        pltpu.sync_copy(items_hbm.at[:, pl.ds(col0, CCOLS)], items_v)
        for s in range(NSUB):
            uid_n, rows_n, sem_n = bufs[(s + 1) % 2]
            if s + 1 < NSUB:
                _issue(user_hbm, score_hbm, uid_n, rows_n, sem_n,
                       col0 + (s + 1) * SUB)
            else:
                @pl.when(g + 1 < NCHUNK)
                def _():
                    _issue(user_hbm, score_hbm, uid_n, rows_n, sem_n,
                           col0 + CCOLS)
            _, rows_c, sem_c = bufs[s % 2]
            _drain(score_hbm, rows_c, sem_c)

            def step(jv, c2, s=s, rows_c=rows_c):
                j = jv >> 1
                half = jv & 1
                c16 = s * SUB + half * L
                rl = half * L + lax.iota(jnp.int32, L)
                t = items_v[j, pl.ds(c16, L)]
                out_v[j, pl.ds(c16, L)] = plsc.load_gather(rows_c, [rl, t])
                return c2

            lax.fori_loop(0, VPS, step, 0)
        pltpu.sync_copy(out_v, out_hbm.at[:, pl.ds(col0, CCOLS)])
        return carry

    lax.fori_loop(0, NCHUNK, chunk, 0)


@jax.jit
def _run(batch_user, items_t, score_mat):
    mesh = plsc.VectorSubcoreMesh(core_axis_name="c", subcore_axis_name="s")
    f = pl.kernel(
        _body,
        out_type=jax.ShapeDtypeStruct((HIST, BATCH), jnp.float32),
        mesh=mesh,
        compiler_params=pltpu.CompilerParams(use_tc_tiling_on_sc=True,
                                             needs_layout_passes=False),
        scratch_types=[
            pltpu.VMEM((SUB,), jnp.int32),
            pltpu.VMEM((SUB,), jnp.int32),
            pltpu.VMEM((HIST, CCOLS), jnp.int32),
            pltpu.VMEM((HIST, CCOLS), jnp.float32),
            pltpu.VMEM((SUB, NUM_ITEMS), jnp.float32),
            pltpu.VMEM((SUB, NUM_ITEMS), jnp.float32),
            pltpu.SemaphoreType.DMA,
            pltpu.SemaphoreType.DMA,
        ],
    )
    return f(batch_user, items_t, score_mat)


def kernel(batch_user, batch_items, score_mat):
    out_t = _run(batch_user, batch_items.T, score_mat)
    return out_t.T
